# Initial kernel scaffold; baseline (speedup 1.0000x reference)
import jax
import jax.numpy as jnp
from jax.experimental import pallas as pl


def kernel(x, edge_index1, pseudo1, edge_index2, pseudo2, W1, root1, bias1, W2, root2, bias2, fc1_w, fc1_b, fc2_w, fc2_b):
    raise NotImplementedError("write your pallas kernel here")


# speedup vs baseline: 3.2154x; 3.2154x over previous
import jax
import jax.numpy as jnp
from jax import lax
from jax.experimental import pallas as pl
from jax.experimental.pallas import tpu as pltpu
from jax.experimental.pallas import tpu_sc as plsc

K = 5
NC, NS, LANES = 2, 16, 16
NTILES = NC * NS

N1 = 64 * 28 * 28
N2 = 64 * 14 * 14
E1 = N1 * 8
E2 = N2 * 8

C1 = 128
C2 = 128
N1H = N1 // 2
N1H_PAD = 25600
EPT1 = E1 // NS
NCH1 = EPT1 // C1
E2_PAD = NTILES * C2 * 25
EPT2 = E2_PAD // NTILES
NCH2 = EPT2 // C2
N2_PAD = 12800

_mesh = plsc.VectorSubcoreMesh(core_axis_name="c", subcore_axis_name="s")
_sc_params = pltpu.CompilerParams(needs_layout_passes=False,
                                  use_tc_tiling_on_sc=False)


def _conv1_edges_body(src_hbm, dst_hbm, p0_hbm, p1_hbm, x_hbm, out_hbm,
                      x_v, srcb, dstb, p0b, p1b, rowbuf, agg):
    c = lax.axis_index("c")
    s = lax.axis_index("s")
    tid = c * NS + s
    zeros16 = jnp.zeros((LANES,), jnp.float32)

    pltpu.sync_copy(x_hbm, x_v)

    def _zero_rowbuf(i, carry):
        rowbuf[i, pl.ds(0, 16)] = zeros16
        rowbuf[i, pl.ds(16, 16)] = zeros16
        return carry
    lax.fori_loop(0, C1, _zero_rowbuf, 0)

    def _zero_agg(i, carry):
        pltpu.sync_copy(rowbuf.at[pl.ds(0, 64)],
                        agg.at[pl.ds(s * 1600 + i * 64, 64)])
        return carry
    lax.fori_loop(0, 25, _zero_agg, 0)
    plsc.subcore_barrier()

    base = s * EPT1
    dst_lo = c * N1H
    iota16 = lax.iota(jnp.int32, 16)
    col_deg = jnp.full((16,), 25, jnp.int32)
    ones16 = jnp.ones((16,), jnp.float32)

    def _chunk(ci, carry):
        off = base + ci * C1
        pltpu.sync_copy(src_hbm.at[pl.ds(off, C1)], srcb)
        pltpu.sync_copy(dst_hbm.at[pl.ds(off, C1)], dstb)
        pltpu.sync_copy(p0_hbm.at[pl.ds(off, C1)], p0b)
        pltpu.sync_copy(p1_hbm.at[pl.ds(off, C1)], p1b)

        def _vec(j, carry2):
            sl = pl.ds(j * LANES, LANES)
            sv = srcb[sl]
            dv = dstb[sl] - dst_lo
            dv = jnp.where((dv >= 0) & (dv < N1H), dv, N1H)
            dstb[sl] = dv
            a0 = p0b[sl] * (K - 1.0)
            a1 = p1b[sl] * (K - 1.0)
            i0 = a0.astype(jnp.int32)
            i1 = a1.astype(jnp.int32)
            fr0 = a0 - i0.astype(jnp.float32)
            fr1 = a1 - i1.astype(jnp.float32)
            xs = plsc.load_gather(x_v, [sv])
            rowv = j * LANES + iota16
            for c0 in (1, 0):
                k0 = jnp.clip(i0 + c0, 0, K - 1)
                w0 = fr0 if c0 else 1.0 - fr0
                for c1 in (1, 0):
                    k1 = jnp.clip(i1 + c1, 0, K - 1)
                    w1 = fr1 if c1 else 1.0 - fr1
                    plsc.store_scatter(rowbuf, [rowv, k0 * K + k1],
                                       (w0 * w1) * xs)
            plsc.store_scatter(rowbuf, [rowv, col_deg], ones16)
            return carry2
        lax.fori_loop(0, C1 // LANES, _vec, 0)

        pltpu.sync_copy(rowbuf, agg.at[dstb], add=True)
        lax.fori_loop(0, C1, _zero_rowbuf, 0)
        return carry
    lax.fori_loop(0, NCH1, _chunk, 0)

    plsc.subcore_barrier()
    pltpu.sync_copy(agg.at[pl.ds(s * 1600, 1600)],
                    out_hbm.at[c, pl.ds(s * 1600, 1600)])


_conv1_edges = pl.kernel(
    _conv1_edges_body,
    out_type=jax.ShapeDtypeStruct((NC, N1H_PAD, 32), jnp.float32),
    mesh=_mesh,
    scratch_types=[
        pltpu.VMEM((N1,), jnp.float32),
        pltpu.VMEM((C1,), jnp.int32),
        pltpu.VMEM((C1,), jnp.int32),
        pltpu.VMEM((C1,), jnp.float32),
        pltpu.VMEM((C1,), jnp.float32),
        pltpu.VMEM((C1, 32), jnp.float32),
        pltpu.VMEM_SHARED((N1H_PAD, 32), jnp.float32),
    ],
    compiler_params=_sc_params,
)


def _conv2_edges_body(src_hbm, dst_hbm, p0_hbm, p1_hbm, t2_hbm, out_hbm,
                      srcb, dstb, p0b, p1b,
                      g0, g1, g2, g3, w0b, w1b, w2b, w3b,
                      r0, r1, r2, r3, msgbuf, agg):
    c = lax.axis_index("c")
    s = lax.axis_index("s")
    tid = c * NS + s
    zeros16 = jnp.zeros((LANES,), jnp.float32)
    iota16 = lax.iota(jnp.int32, 16)

    def _zero_msgbuf(i, carry):
        for hh in range(5):
            msgbuf[i, pl.ds(hh * 16, 16)] = zeros16
        return carry
    lax.fori_loop(0, C2, _zero_msgbuf, 0)

    def _zero_agg(i, carry):
        pltpu.sync_copy(msgbuf.at[pl.ds(0, 80)],
                        agg.at[pl.ds(s * 800 + i * 80, 80)])
        return carry
    lax.fori_loop(0, 10, _zero_agg, 0)
    plsc.subcore_barrier()

    deg16 = jnp.where(iota16 == 0, 1.0, 0.0).astype(jnp.float32)

    def _init_deg(i, carry):
        msgbuf[i, pl.ds(64, 16)] = deg16
        return carry
    lax.fori_loop(0, C2, _init_deg, 0)

    base = tid * EPT2

    def _chunk(ci, carry):
        off = base + ci * C2
        pltpu.sync_copy(src_hbm.at[pl.ds(off, C2)], srcb)
        pltpu.sync_copy(dst_hbm.at[pl.ds(off, C2)], dstb)
        pltpu.sync_copy(p0_hbm.at[pl.ds(off, C2)], p0b)
        pltpu.sync_copy(p1_hbm.at[pl.ds(off, C2)], p1b)

        def _vec(j, carry2):
            sl = pl.ds(j * LANES, LANES)
            sv = srcb[sl]
            a0 = p0b[sl] * (K - 1.0)
            a1 = p1b[sl] * (K - 1.0)
            i0 = a0.astype(jnp.int32)
            i1 = a1.astype(jnp.int32)
            fr0 = a0 - i0.astype(jnp.float32)
            fr1 = a1 - i1.astype(jnp.float32)
            sv25 = sv * 25
            gb = (g0, g1, g2, g3)
            wb = (w0b, w1b, w2b, w3b)
            ix = 0
            for c0 in (0, 1):
                k0 = jnp.clip(i0 + c0, 0, K - 1)
                w0 = fr0 if c0 else 1.0 - fr0
                for c1 in (0, 1):
                    k1 = jnp.clip(i1 + c1, 0, K - 1)
                    w1 = fr1 if c1 else 1.0 - fr1
                    gb[ix][sl] = sv25 + k0 * K + k1
                    wb[ix][sl] = w0 * w1
                    ix += 1
            return carry2
        lax.fori_loop(0, C2 // LANES, _vec, 0)

        pltpu.sync_copy(t2_hbm.at[g0], r0)
        pltpu.sync_copy(t2_hbm.at[g1], r1)
        pltpu.sync_copy(t2_hbm.at[g2], r2)
        pltpu.sync_copy(t2_hbm.at[g3], r3)

        def _mgrp(j, carry2):
            ev = j * LANES + iota16
            slw = pl.ds(j * LANES, LANES)
            wv0 = w0b[slw]
            wv1 = w1b[slw]
            wv2 = w2b[slw]
            wv3 = w3b[slw]

            def _col(oo, carry3):
                for o4 in range(4):
                    ov = jnp.full((16,), oo * 4 + o4, jnp.int32)
                    m = (wv0 * plsc.load_gather(r0, [ev, ov])
                         + wv1 * plsc.load_gather(r1, [ev, ov])
                         + wv2 * plsc.load_gather(r2, [ev, ov])
                         + wv3 * plsc.load_gather(r3, [ev, ov]))
                    plsc.store_scatter(msgbuf, [ev, ov], m)
                return carry3
            lax.fori_loop(0, 16, _col, 0)
            return carry2
        lax.fori_loop(0, C2 // LANES, _mgrp, 0)

        pltpu.sync_copy(msgbuf, agg.at[dstb], add=True)
        return carry
    lax.fori_loop(0, NCH2, _chunk, 0)

    plsc.subcore_barrier()
    pltpu.sync_copy(agg.at[pl.ds(s * 800, 800)],
                    out_hbm.at[c, pl.ds(s * 800, 800)])


_conv2_edges = pl.kernel(
    _conv2_edges_body,
    out_type=jax.ShapeDtypeStruct((NC, N2_PAD, 80), jnp.float32),
    mesh=_mesh,
    scratch_types=[
        pltpu.VMEM((C2,), jnp.int32),
        pltpu.VMEM((C2,), jnp.int32),
        pltpu.VMEM((C2,), jnp.float32),
        pltpu.VMEM((C2,), jnp.float32),
        pltpu.VMEM((C2,), jnp.int32),
        pltpu.VMEM((C2,), jnp.int32),
        pltpu.VMEM((C2,), jnp.int32),
        pltpu.VMEM((C2,), jnp.int32),
        pltpu.VMEM((C2,), jnp.float32),
        pltpu.VMEM((C2,), jnp.float32),
        pltpu.VMEM((C2,), jnp.float32),
        pltpu.VMEM((C2,), jnp.float32),
        pltpu.VMEM((C2, 64), jnp.float32),
        pltpu.VMEM((C2, 64), jnp.float32),
        pltpu.VMEM((C2, 64), jnp.float32),
        pltpu.VMEM((C2, 64), jnp.float32),
        pltpu.VMEM((C2, 80), jnp.float32),
        pltpu.VMEM_SHARED((N2_PAD, 80), jnp.float32),
    ],
    compiler_params=_sc_params,
)


def _elu(v):
    return jnp.where(v > 0, v, jnp.exp(jnp.minimum(v, 0.0)) - 1.0)


def _tca1_body(acc_ref, x_ref, w1r_ref, root1_ref, bias1_ref, h_ref):
    a = acc_ref[0]
    deg = jnp.maximum(a[:, 25:26], 1.0)
    out = jnp.dot(a[:, :25], w1r_ref[...],
                  preferred_element_type=jnp.float32) / deg
    out = out + x_ref[...] * root1_ref[...] + bias1_ref[...][None, :]
    out = _elu(out)
    p = out.reshape(14, 2, 14, 2, 32).max(axis=(1, 3))
    h_ref[...] = p.reshape(1, 196, 32)


def _tca2_body(h_ref, w_ref, o_ref):
    o_ref[...] = jnp.dot(h_ref[...], w_ref[...],
                         preferred_element_type=jnp.float32)


def _tcb_body(acc_ref, h_ref, root2_ref, bias2_ref, o_ref):
    a = acc_ref[0, :N2] + acc_ref[1, :N2]
    deg = jnp.maximum(a[:, 64:65], 1.0)
    out = a[:, :64] / deg
    out = out + jnp.dot(h_ref[...], root2_ref[...],
                        preferred_element_type=jnp.float32)
    out = _elu(out + bias2_ref[...][None, :])
    p = out.reshape(64, 7, 2, 7, 2, 64).max(axis=(2, 4))
    o_ref[...] = p.reshape(3136, 64)


def _tcc_body(inp_ref, w1_ref, b1_ref, w2_ref, b2_ref, o_ref):
    z = jnp.dot(inp_ref[...], w1_ref[...],
                preferred_element_type=jnp.float32) + b1_ref[...][None, :]
    z = _elu(z)
    z = jnp.dot(z, w2_ref[...],
                preferred_element_type=jnp.float32) + b2_ref[...][None, :]
    z = _elu(z)
    m = jnp.max(z, axis=-1, keepdims=True)
    lse = m + jnp.log(jnp.sum(jnp.exp(z - m), axis=-1, keepdims=True))
    o_ref[...] = z - lse


def kernel(x, edge_index1, pseudo1, edge_index2, pseudo2,
           W1, root1, bias1, W2, root2, bias2, fc1_w, fc1_b, fc2_w, fc2_b):
    f32 = jnp.float32
    xf = x[:, 0]
    src1 = edge_index1[0].astype(jnp.int32)
    dst1 = edge_index1[1].astype(jnp.int32)
    p01 = pseudo1[:, 0]
    p11 = pseudo1[:, 1]

    acc1 = _conv1_edges(src1, dst1, p01, p11, xf)

    h = pl.pallas_call(
        _tca1_body,
        grid=(64,),
        in_specs=[
            pl.BlockSpec((1, 784, 32), lambda i: (i // 32, i % 32, 0)),
            pl.BlockSpec((784, 1), lambda i: (i, 0)),
            pl.BlockSpec((25, 32), lambda i: (0, 0)),
            pl.BlockSpec((1, 32), lambda i: (0, 0)),
            pl.BlockSpec((32,), lambda i: (0,)),
        ],
        out_specs=pl.BlockSpec((1, 196, 32), lambda i: (i, 0, 0)),
        out_shape=jax.ShapeDtypeStruct((64, 196, 32), f32),
    )(acc1, x, W1[:, 0, :], root1, bias1)
    h = h.reshape(N2, 32)

    W2f = W2.transpose(1, 0, 2).reshape(32, 25 * 64)
    t2 = pl.pallas_call(
        _tca2_body,
        grid=(49,),
        in_specs=[
            pl.BlockSpec((256, 32), lambda i: (i, 0)),
            pl.BlockSpec((32, 1600), lambda i: (0, 0)),
        ],
        out_specs=pl.BlockSpec((256, 1600), lambda i: (i, 0)),
        out_shape=jax.ShapeDtypeStruct((N2, 1600), f32),
    )(h, W2f)
    t2 = t2.reshape(N2 * 25, 64)

    npad = E2_PAD - E2
    src2 = jnp.concatenate([edge_index2[0].astype(jnp.int32),
                            jnp.zeros((npad,), jnp.int32)])
    dst2 = jnp.concatenate([edge_index2[1].astype(jnp.int32),
                            jnp.full((npad,), N2, jnp.int32)])
    p02 = jnp.concatenate([pseudo2[:, 0], jnp.zeros((npad,), f32)])
    p12 = jnp.concatenate([pseudo2[:, 1], jnp.zeros((npad,), f32)])

    acc2 = _conv2_edges(src2, dst2, p02, p12, t2)

    pooled = pl.pallas_call(
        _tcb_body,
        in_specs=[
            pl.BlockSpec((2, N2_PAD, 80), lambda: (0, 0, 0)),
            pl.BlockSpec((N2, 32), lambda: (0, 0)),
            pl.BlockSpec((32, 64), lambda: (0, 0)),
            pl.BlockSpec((64,), lambda: (0,)),
        ],
        out_specs=pl.BlockSpec((3136, 64), lambda: (0, 0)),
        out_shape=jax.ShapeDtypeStruct((3136, 64), f32),
    )(acc2, h, root2, bias2)

    out = pl.pallas_call(
        _tcc_body,
        in_specs=[
            pl.BlockSpec((64, 3136), lambda: (0, 0)),
            pl.BlockSpec((3136, 512), lambda: (0, 0)),
            pl.BlockSpec((512,), lambda: (0,)),
            pl.BlockSpec((512, 10), lambda: (0, 0)),
            pl.BlockSpec((10,), lambda: (0,)),
        ],
        out_specs=pl.BlockSpec((64, 10), lambda: (0, 0)),
        out_shape=jax.ShapeDtypeStruct((64, 10), f32),
    )(pooled.reshape(64, 3136), fc1_w, fc1_b, fc2_w, fc2_b)
    return out


# speedup vs baseline: 4.0723x; 1.2665x over previous
import jax
import jax.numpy as jnp
from jax import lax
from jax.experimental import pallas as pl
from jax.experimental.pallas import tpu as pltpu
from jax.experimental.pallas import tpu_sc as plsc

K = 5
NC, NS, LANES = 2, 16, 16
NTILES = NC * NS

N1 = 64 * 28 * 28
N2 = 64 * 14 * 14
E1 = N1 * 8
E2 = N2 * 8

C1 = 128
C2 = 128
N1H = N1 // 2
N1H_PAD = 25600
EPT1 = E1 // NS
NCH1 = EPT1 // C1
E2_PAD = NTILES * C2 * 25
EPT2 = E2_PAD // NTILES
NCH2 = EPT2 // C2
N2_PAD = 12800

_mesh = plsc.VectorSubcoreMesh(core_axis_name="c", subcore_axis_name="s")
_sc_params = pltpu.CompilerParams(needs_layout_passes=False,
                                  use_tc_tiling_on_sc=False)


def _conv1_edges_body(ed_hbm, x_hbm, out_hbm,
                      x_v, ebuf, dstb, rowbuf, agg):
    c = lax.axis_index("c")
    s = lax.axis_index("s")
    tid = c * NS + s
    zeros16 = jnp.zeros((LANES,), jnp.float32)

    pltpu.sync_copy(x_hbm, x_v)

    def _zero_rowbuf(i, carry):
        rowbuf[i, pl.ds(0, 16)] = zeros16
        rowbuf[i, pl.ds(16, 16)] = zeros16
        return carry
    lax.fori_loop(0, C1, _zero_rowbuf, 0)

    def _zero_agg(i, carry):
        pltpu.sync_copy(rowbuf.at[pl.ds(0, 64)],
                        agg.at[pl.ds(s * 1600 + i * 64, 64)])
        return carry
    lax.fori_loop(0, 25, _zero_agg, 0)
    plsc.subcore_barrier()

    cbase = s * NCH1
    dst_lo = c * N1H
    iota16 = lax.iota(jnp.int32, 16)
    col_deg = jnp.full((16,), 25, jnp.int32)
    ones16 = jnp.ones((16,), jnp.float32)

    def _chunk(ci, carry):
        pltpu.sync_copy(ed_hbm.at[cbase + ci], ebuf)

        def _vec(j, carry2):
            sl = pl.ds(j * LANES, LANES)
            sv = ebuf[0, sl]
            dv = ebuf[1, sl] - dst_lo
            dv = jnp.where((dv >= 0) & (dv < N1H), dv, N1H)
            dstb[sl] = dv
            a0 = plsc.bitcast(ebuf[2, sl], jnp.float32) * (K - 1.0)
            a1 = plsc.bitcast(ebuf[3, sl], jnp.float32) * (K - 1.0)
            i0 = a0.astype(jnp.int32)
            i1 = a1.astype(jnp.int32)
            fr0 = a0 - i0.astype(jnp.float32)
            fr1 = a1 - i1.astype(jnp.float32)
            xs = plsc.load_gather(x_v, [sv])
            rowv = j * LANES + iota16
            for c0 in (1, 0):
                k0 = jnp.clip(i0 + c0, 0, K - 1)
                w0 = fr0 if c0 else 1.0 - fr0
                for c1 in (1, 0):
                    k1 = jnp.clip(i1 + c1, 0, K - 1)
                    w1 = fr1 if c1 else 1.0 - fr1
                    plsc.store_scatter(rowbuf, [rowv, k0 * K + k1],
                                       (w0 * w1) * xs)
            plsc.store_scatter(rowbuf, [rowv, col_deg], ones16)
            return carry2
        lax.fori_loop(0, C1 // LANES, _vec, 0)

        pltpu.sync_copy(rowbuf, agg.at[dstb], add=True)
        lax.fori_loop(0, C1, _zero_rowbuf, 0)
        return carry
    lax.fori_loop(0, NCH1, _chunk, 0)

    plsc.subcore_barrier()
    pltpu.sync_copy(agg.at[pl.ds(s * 1600, 1600)],
                    out_hbm.at[c, pl.ds(s * 1600, 1600)])


_conv1_edges = pl.kernel(
    _conv1_edges_body,
    out_type=jax.ShapeDtypeStruct((NC, N1H_PAD, 32), jnp.float32),
    mesh=_mesh,
    scratch_types=[
        pltpu.VMEM((N1,), jnp.float32),
        pltpu.VMEM((4, C1), jnp.int32),
        pltpu.VMEM((C1,), jnp.int32),
        pltpu.VMEM((C1, 32), jnp.float32),
        pltpu.VMEM_SHARED((N1H_PAD, 32), jnp.float32),
    ],
    compiler_params=_sc_params,
)


def _conv2_edges_body(ed_hbm, t2_hbm, out_hbm,
                      ebuf, dstb,
                      g0, g1, g2, g3, w0b, w1b, w2b, w3b,
                      r0, r1, r2, r3, msgbuf, agg, gsem):
    c = lax.axis_index("c")
    s = lax.axis_index("s")
    tid = c * NS + s
    zeros16 = jnp.zeros((LANES,), jnp.float32)
    iota16 = lax.iota(jnp.int32, 16)

    def _zero_msgbuf(i, carry):
        for hh in range(5):
            msgbuf[i, pl.ds(hh * 16, 16)] = zeros16
        return carry
    lax.fori_loop(0, C2, _zero_msgbuf, 0)

    def _zero_agg(i, carry):
        pltpu.sync_copy(msgbuf.at[pl.ds(0, 80)],
                        agg.at[pl.ds(s * 800 + i * 80, 80)])
        return carry
    lax.fori_loop(0, 10, _zero_agg, 0)
    plsc.subcore_barrier()

    deg16 = jnp.where(iota16 == 0, 1.0, 0.0).astype(jnp.float32)

    def _init_deg(i, carry):
        msgbuf[i, pl.ds(64, 16)] = deg16
        return carry
    lax.fori_loop(0, C2, _init_deg, 0)

    cbase = tid * NCH2

    def _chunk(ci, carry):
        pltpu.sync_copy(ed_hbm.at[cbase + ci], ebuf)

        def _vec(j, carry2):
            sl = pl.ds(j * LANES, LANES)
            sv = ebuf[0, sl]
            dstb[sl] = ebuf[1, sl]
            a0 = plsc.bitcast(ebuf[2, sl], jnp.float32) * (K - 1.0)
            a1 = plsc.bitcast(ebuf[3, sl], jnp.float32) * (K - 1.0)
            i0 = a0.astype(jnp.int32)
            i1 = a1.astype(jnp.int32)
            fr0 = a0 - i0.astype(jnp.float32)
            fr1 = a1 - i1.astype(jnp.float32)
            sv25 = sv * 25
            gb = (g0, g1, g2, g3)
            wb = (w0b, w1b, w2b, w3b)
            ix = 0
            for c0 in (0, 1):
                k0 = jnp.clip(i0 + c0, 0, K - 1)
                w0 = fr0 if c0 else 1.0 - fr0
                for c1 in (0, 1):
                    k1 = jnp.clip(i1 + c1, 0, K - 1)
                    w1 = fr1 if c1 else 1.0 - fr1
                    gb[ix][sl] = sv25 + k0 * K + k1
                    wb[ix][sl] = w0 * w1
                    ix += 1
            return carry2
        lax.fori_loop(0, C2 // LANES, _vec, 0)

        cp0 = pltpu.async_copy(t2_hbm.at[g0], r0, gsem)
        cp1 = pltpu.async_copy(t2_hbm.at[g1], r1, gsem)
        cp2 = pltpu.async_copy(t2_hbm.at[g2], r2, gsem)
        cp3 = pltpu.async_copy(t2_hbm.at[g3], r3, gsem)
        cp0.wait()
        cp1.wait()
        cp2.wait()
        cp3.wait()

        def _mgrp(j, carry2):
            ev = j * LANES + iota16
            slw = pl.ds(j * LANES, LANES)
            wv0 = w0b[slw]
            wv1 = w1b[slw]
            wv2 = w2b[slw]
            wv3 = w3b[slw]

            def _col(oo, carry3):
                for o4 in range(4):
                    ov = jnp.full((16,), oo * 4 + o4, jnp.int32)
                    m = (wv0 * plsc.load_gather(r0, [ev, ov])
                         + wv1 * plsc.load_gather(r1, [ev, ov])
                         + wv2 * plsc.load_gather(r2, [ev, ov])
                         + wv3 * plsc.load_gather(r3, [ev, ov]))
                    plsc.store_scatter(msgbuf, [ev, ov], m)
                return carry3
            lax.fori_loop(0, 16, _col, 0)
            return carry2
        lax.fori_loop(0, C2 // LANES, _mgrp, 0)

        pltpu.sync_copy(msgbuf, agg.at[dstb], add=True)
        return carry
    lax.fori_loop(0, NCH2, _chunk, 0)

    plsc.subcore_barrier()
    pltpu.sync_copy(agg.at[pl.ds(s * 800, 800)],
                    out_hbm.at[c, pl.ds(s * 800, 800)])


_conv2_edges = pl.kernel(
    _conv2_edges_body,
    out_type=jax.ShapeDtypeStruct((NC, N2_PAD, 80), jnp.float32),
    mesh=_mesh,
    scratch_types=[
        pltpu.VMEM((4, C2), jnp.int32),
        pltpu.VMEM((C2,), jnp.int32),
        pltpu.VMEM((C2,), jnp.int32),
        pltpu.VMEM((C2,), jnp.int32),
        pltpu.VMEM((C2,), jnp.int32),
        pltpu.VMEM((C2,), jnp.int32),
        pltpu.VMEM((C2,), jnp.float32),
        pltpu.VMEM((C2,), jnp.float32),
        pltpu.VMEM((C2,), jnp.float32),
        pltpu.VMEM((C2,), jnp.float32),
        pltpu.VMEM((C2, 64), jnp.float32),
        pltpu.VMEM((C2, 64), jnp.float32),
        pltpu.VMEM((C2, 64), jnp.float32),
        pltpu.VMEM((C2, 64), jnp.float32),
        pltpu.VMEM((C2, 80), jnp.float32),
        pltpu.VMEM_SHARED((N2_PAD, 80), jnp.float32),
        pltpu.SemaphoreType.DMA,
    ],
    compiler_params=_sc_params,
)


def _elu(v):
    return jnp.where(v > 0, v, jnp.exp(jnp.minimum(v, 0.0)) - 1.0)


def _tca1_body(acc_ref, x_ref, w1r_ref, root1_ref, bias1_ref, h_ref):
    a = acc_ref[0]
    deg = jnp.maximum(a[:, 25:26], 1.0)
    out = jnp.dot(a[:, :25], w1r_ref[...],
                  preferred_element_type=jnp.float32) / deg
    out = out + x_ref[...] * root1_ref[...] + bias1_ref[...][None, :]
    out = _elu(out)
    p = out.reshape(14, 2, 14, 2, 32).max(axis=(1, 3))
    h_ref[...] = p.reshape(1, 196, 32)


def _tca2_body(h_ref, w_ref, o_ref):
    o_ref[...] = jnp.dot(h_ref[...], w_ref[...],
                         preferred_element_type=jnp.float32)


def _tcb_body(acc_ref, h_ref, root2_ref, bias2_ref, o_ref):
    a = acc_ref[0, :N2] + acc_ref[1, :N2]
    deg = jnp.maximum(a[:, 64:65], 1.0)
    out = a[:, :64] / deg
    out = out + jnp.dot(h_ref[...], root2_ref[...],
                        preferred_element_type=jnp.float32)
    out = _elu(out + bias2_ref[...][None, :])
    p = out.reshape(64, 7, 2, 7, 2, 64).max(axis=(2, 4))
    o_ref[...] = p.reshape(3136, 64)


def _tcc_body(inp_ref, w1_ref, b1_ref, w2_ref, b2_ref, o_ref):
    z = jnp.dot(inp_ref[...], w1_ref[...],
                preferred_element_type=jnp.float32) + b1_ref[...][None, :]
    z = _elu(z)
    z = jnp.dot(z, w2_ref[...],
                preferred_element_type=jnp.float32) + b2_ref[...][None, :]
    z = _elu(z)
    m = jnp.max(z, axis=-1, keepdims=True)
    lse = m + jnp.log(jnp.sum(jnp.exp(z - m), axis=-1, keepdims=True))
    o_ref[...] = z - lse


def _pack_edges(ei, ps):
    src = ei[0].astype(jnp.int32)
    dst = ei[1].astype(jnp.int32)
    p0 = lax.bitcast_convert_type(ps[:, 0], jnp.int32)
    p1 = lax.bitcast_convert_type(ps[:, 1], jnp.int32)
    n = src.shape[0] // 128
    return jnp.stack([src.reshape(n, 128), dst.reshape(n, 128),
                      p0.reshape(n, 128), p1.reshape(n, 128)], axis=1)


def kernel(x, edge_index1, pseudo1, edge_index2, pseudo2,
           W1, root1, bias1, W2, root2, bias2, fc1_w, fc1_b, fc2_w, fc2_b):
    f32 = jnp.float32
    xf = x[:, 0]

    acc1 = _conv1_edges(_pack_edges(edge_index1, pseudo1), xf)

    h = pl.pallas_call(
        _tca1_body,
        grid=(64,),
        in_specs=[
            pl.BlockSpec((1, 784, 32), lambda i: (i // 32, i % 32, 0)),
            pl.BlockSpec((784, 1), lambda i: (i, 0)),
            pl.BlockSpec((25, 32), lambda i: (0, 0)),
            pl.BlockSpec((1, 32), lambda i: (0, 0)),
            pl.BlockSpec((32,), lambda i: (0,)),
        ],
        out_specs=pl.BlockSpec((1, 196, 32), lambda i: (i, 0, 0)),
        out_shape=jax.ShapeDtypeStruct((64, 196, 32), f32),
    )(acc1, x, W1[:, 0, :], root1, bias1)
    h = h.reshape(N2, 32)

    W2f = W2.transpose(1, 0, 2).reshape(32, 25 * 64)
    t2 = pl.pallas_call(
        _tca2_body,
        grid=(49,),
        in_specs=[
            pl.BlockSpec((256, 32), lambda i: (i, 0)),
            pl.BlockSpec((32, 1600), lambda i: (0, 0)),
        ],
        out_specs=pl.BlockSpec((256, 1600), lambda i: (i, 0)),
        out_shape=jax.ShapeDtypeStruct((N2, 1600), f32),
    )(h, W2f)
    t2 = t2.reshape(N2 * 25, 64)

    npad = E2_PAD - E2
    ei2 = jnp.concatenate(
        [edge_index2.astype(jnp.int32),
         jnp.stack([jnp.zeros((npad,), jnp.int32),
                    jnp.full((npad,), N2, jnp.int32)])], axis=1)
    ps2 = jnp.concatenate([pseudo2, jnp.zeros((npad, 2), f32)], axis=0)

    acc2 = _conv2_edges(_pack_edges(ei2, ps2), t2)

    pooled = pl.pallas_call(
        _tcb_body,
        in_specs=[
            pl.BlockSpec((2, N2_PAD, 80), lambda: (0, 0, 0)),
            pl.BlockSpec((N2, 32), lambda: (0, 0)),
            pl.BlockSpec((32, 64), lambda: (0, 0)),
            pl.BlockSpec((64,), lambda: (0,)),
        ],
        out_specs=pl.BlockSpec((3136, 64), lambda: (0, 0)),
        out_shape=jax.ShapeDtypeStruct((3136, 64), f32),
    )(acc2, h, root2, bias2)

    out = pl.pallas_call(
        _tcc_body,
        in_specs=[
            pl.BlockSpec((64, 3136), lambda: (0, 0)),
            pl.BlockSpec((3136, 512), lambda: (0, 0)),
            pl.BlockSpec((512,), lambda: (0,)),
            pl.BlockSpec((512, 10), lambda: (0, 0)),
            pl.BlockSpec((10,), lambda: (0,)),
        ],
        out_specs=pl.BlockSpec((64, 10), lambda: (0, 0)),
        out_shape=jax.ShapeDtypeStruct((64, 10), f32),
    )(pooled.reshape(64, 3136), fc1_w, fc1_b, fc2_w, fc2_b)
    return out


# speedup vs baseline: 5.0142x; 1.2313x over previous
import jax
import jax.numpy as jnp
from jax import lax
from jax.experimental import pallas as pl
from jax.experimental.pallas import tpu as pltpu
from jax.experimental.pallas import tpu_sc as plsc

K = 5
NC, NS, LANES = 2, 16, 16
NTILES = NC * NS

N1 = 64 * 28 * 28
N2 = 64 * 14 * 14
E1 = N1 * 8
E2 = N2 * 8

C1 = 128
C2 = 64
N1H = N1 // 2
N1H_PAD = 25600
EPT1 = E1 // NS
NCH1 = EPT1 // C1
EPT2 = E2 // NTILES
NCH2 = EPT2 // C2
N2_PAD = 12800

_mesh = plsc.VectorSubcoreMesh(core_axis_name="c", subcore_axis_name="s")
_sc_params = pltpu.CompilerParams(needs_layout_passes=False,
                                  use_tc_tiling_on_sc=False)


def _conv1_edges_body(ed_hbm, x_hbm, out_hbm,
                      x_v, ebuf, dstb, rowbuf, agg, esem, ssem):
    c = lax.axis_index("c")
    s = lax.axis_index("s")
    zeros16 = jnp.zeros((LANES,), jnp.float32)
    cbase = s * NCH1
    maxc = NS * NCH1 - 1
    dst_lo = c * N1H
    iota16 = lax.iota(jnp.int32, 16)
    col_deg = jnp.full((16,), 25, jnp.int32)
    ones16 = jnp.ones((16,), jnp.float32)

    def _ed_issue(cidx, b):
        pltpu.async_copy(ed_hbm.at[jnp.minimum(cbase + cidx, maxc)],
                         ebuf.at[b], esem)

    def _ed_wait(b):
        pltpu.make_async_copy(ed_hbm.at[0], ebuf.at[b], esem).wait()

    def _s_issue(b):
        pltpu.async_copy(rowbuf.at[b], agg.at[dstb.at[b]], ssem, add=True)

    def _s_wait(b):
        pltpu.make_async_copy(rowbuf.at[b], agg.at[dstb.at[b]], ssem).wait()

    _ed_issue(0, 0)
    pltpu.sync_copy(x_hbm, x_v)

    def _zero_rowbuf(b):
        def _zr(i, carry):
            for r4 in range(4):
                rowbuf[b, i * 4 + r4, pl.ds(0, 16)] = zeros16
                rowbuf[b, i * 4 + r4, pl.ds(16, 16)] = zeros16
            return carry
        lax.fori_loop(0, C1 // 4, _zr, 0)

    _zero_rowbuf(0)
    _zero_rowbuf(1)

    def _zero_agg(i, carry):
        pltpu.sync_copy(rowbuf.at[0, pl.ds(0, 64)],
                        agg.at[pl.ds(s * 1600 + i * 64, 64)])
        return carry
    lax.fori_loop(0, 25, _zero_agg, 0)
    plsc.subcore_barrier()

    def _do_v(b):
        def _vec(j, carry):
            sl = pl.ds(j * LANES, LANES)
            sv = ebuf[b, 0, sl]
            dv = ebuf[b, 1, sl] - dst_lo
            dv = jnp.where((dv >= 0) & (dv < N1H), dv, N1H)
            dstb[b, sl] = dv
            a0 = plsc.bitcast(ebuf[b, 2, sl], jnp.float32) * (K - 1.0)
            a1 = plsc.bitcast(ebuf[b, 3, sl], jnp.float32) * (K - 1.0)
            i0 = a0.astype(jnp.int32)
            i1 = a1.astype(jnp.int32)
            fr0 = a0 - i0.astype(jnp.float32)
            fr1 = a1 - i1.astype(jnp.float32)
            xs = plsc.load_gather(x_v, [sv])
            rowv = j * LANES + iota16
            rb = rowbuf.at[b]
            for c0 in (1, 0):
                k0 = jnp.clip(i0 + c0, 0, K - 1)
                w0 = fr0 if c0 else 1.0 - fr0
                for c1 in (1, 0):
                    k1 = jnp.clip(i1 + c1, 0, K - 1)
                    w1 = fr1 if c1 else 1.0 - fr1
                    plsc.store_scatter(rb, [rowv, k0 * K + k1],
                                       (w0 * w1) * xs)
            plsc.store_scatter(rb, [rowv, col_deg], ones16)
            return carry
        lax.fori_loop(0, C1 // LANES, _vec, 0)

    _ed_wait(0)
    _ed_issue(1, 1)
    _do_v(0)
    _ed_issue(2, 0)
    _s_issue(0)
    _ed_wait(1)
    _do_v(1)
    _ed_issue(3, 1)
    _s_issue(1)

    def _main(k, carry):
        for half in (0, 1):
            ci = 2 + 2 * k + half
            b = half
            _ed_wait(b)
            _s_wait(b)
            _zero_rowbuf(b)
            _do_v(b)
            _ed_issue(ci + 2, b)
            _s_issue(b)
        return carry
    lax.fori_loop(0, (NCH1 - 2) // 2, _main, 0)

    _s_wait(0)
    _s_wait(1)
    _ed_wait(0)
    _ed_wait(1)

    plsc.subcore_barrier()
    pltpu.sync_copy(agg.at[pl.ds(s * 1600, 1600)],
                    out_hbm.at[c, pl.ds(s * 1600, 1600)])


_conv1_edges = pl.kernel(
    _conv1_edges_body,
    out_type=jax.ShapeDtypeStruct((NC, N1H_PAD, 32), jnp.float32),
    mesh=_mesh,
    scratch_types=[
        pltpu.VMEM((N1,), jnp.float32),
        pltpu.VMEM((2, 4, C1), jnp.int32),
        pltpu.VMEM((2, C1), jnp.int32),
        pltpu.VMEM((2, C1, 32), jnp.float32),
        pltpu.VMEM_SHARED((N1H_PAD, 32), jnp.float32),
        pltpu.SemaphoreType.DMA,
        pltpu.SemaphoreType.DMA,
    ],
    compiler_params=_sc_params,
)


def _conv2_edges_body(ed_hbm, t2_hbm, out_hbm,
                      ebuf, dstb, gbuf, wbuf, rbuf, msgbuf, agg, gsem, esem):
    c = lax.axis_index("c")
    s = lax.axis_index("s")
    tid = c * NS + s
    zeros16 = jnp.zeros((LANES,), jnp.float32)
    iota16 = lax.iota(jnp.int32, 16)
    cbase = tid * NCH2
    maxc = NTILES * NCH2 - 1

    def _zero_msgbuf(i, carry):
        for hh in range(5):
            msgbuf[i, pl.ds(hh * 16, 16)] = zeros16
        return carry
    lax.fori_loop(0, C2, _zero_msgbuf, 0)

    def _zero_agg(i, carry):
        pltpu.sync_copy(msgbuf.at[pl.ds(0, 50)],
                        agg.at[pl.ds(s * 800 + i * 50, 50)])
        return carry
    lax.fori_loop(0, 16, _zero_agg, 0)
    plsc.subcore_barrier()

    deg16 = jnp.where(iota16 == 0, 1.0, 0.0).astype(jnp.float32)

    def _init_deg(i, carry):
        msgbuf[i, pl.ds(64, 16)] = deg16
        return carry
    lax.fori_loop(0, C2, _init_deg, 0)

    def _ed_issue(cidx, b):
        pltpu.async_copy(ed_hbm.at[jnp.minimum(cbase + cidx, maxc)],
                         ebuf.at[b], esem)

    def _ed_wait(b):
        pltpu.make_async_copy(ed_hbm.at[0], ebuf.at[b], esem).wait()

    def _do_v(b):
        def _vec(j, carry):
            sl = pl.ds(j * LANES, LANES)
            sv = ebuf[b, 0, sl]
            dstb[b, sl] = ebuf[b, 1, sl]
            a0 = plsc.bitcast(ebuf[b, 2, sl], jnp.float32) * (K - 1.0)
            a1 = plsc.bitcast(ebuf[b, 3, sl], jnp.float32) * (K - 1.0)
            i0 = a0.astype(jnp.int32)
            i1 = a1.astype(jnp.int32)
            fr0 = a0 - i0.astype(jnp.float32)
            fr1 = a1 - i1.astype(jnp.float32)
            sv25 = sv * 25
            ix = 0
            for c0 in (0, 1):
                k0 = jnp.clip(i0 + c0, 0, K - 1)
                w0 = fr0 if c0 else 1.0 - fr0
                for c1 in (0, 1):
                    k1 = jnp.clip(i1 + c1, 0, K - 1)
                    w1 = fr1 if c1 else 1.0 - fr1
                    gbuf[b, ix, sl] = sv25 + k0 * K + k1
                    wbuf[b, ix, sl] = w0 * w1
                    ix += 1
            return carry
        lax.fori_loop(0, C2 // LANES, _vec, 0)

    def _g_issue(b):
        for cc in range(4):
            pltpu.async_copy(t2_hbm.at[gbuf.at[b, cc]],
                             rbuf.at[b, pl.ds(cc * C2, C2)], gsem)

    def _g_wait(b):
        for cc in range(4):
            pltpu.make_async_copy(t2_hbm.at[gbuf.at[b, cc]],
                                  rbuf.at[b, pl.ds(cc * C2, C2)], gsem).wait()

    def _do_ms(b):
        rb = rbuf.at[b]

        def _mgrp(j, carry):
            ev = j * LANES + iota16
            slw = pl.ds(j * LANES, LANES)
            wv0 = wbuf[b, 0, slw]
            wv1 = wbuf[b, 1, slw]
            wv2 = wbuf[b, 2, slw]
            wv3 = wbuf[b, 3, slw]

            def _col(oo, carry3):
                for o4 in range(4):
                    ov = jnp.full((16,), oo * 4 + o4, jnp.int32)
                    m = (wv0 * plsc.load_gather(rb, [ev, ov])
                         + wv1 * plsc.load_gather(rb, [ev + C2, ov])
                         + wv2 * plsc.load_gather(rb, [ev + 2 * C2, ov])
                         + wv3 * plsc.load_gather(rb, [ev + 3 * C2, ov]))
                    plsc.store_scatter(msgbuf, [ev, ov], m)
                return carry3
            lax.fori_loop(0, 16, _col, 0)
            return carry
        lax.fori_loop(0, C2 // LANES, _mgrp, 0)
        pltpu.sync_copy(msgbuf, agg.at[dstb.at[b]], add=True)

    _ed_issue(0, 0)
    _ed_wait(0)
    _do_v(0)
    _g_issue(0)
    _ed_issue(1, 1)

    def _main(k, carry):
        for half in (0, 1):
            ci = 1 + 2 * k + half
            b = 1 - half
            _ed_wait(b)
            _do_v(b)
            _g_issue(b)
            _ed_issue(ci + 1, 1 - b)
            _g_wait(1 - b)
            _do_ms(1 - b)
        return carry
    lax.fori_loop(0, NCH2 // 2, _main, 0)

    _g_wait(0)
    _do_ms(0)
    _ed_wait(1)

    plsc.subcore_barrier()
    pltpu.sync_copy(agg.at[pl.ds(s * 800, 800)],
                    out_hbm.at[c, pl.ds(s * 800, 800)])


_conv2_edges = pl.kernel(
    _conv2_edges_body,
    out_type=jax.ShapeDtypeStruct((NC, N2_PAD, 80), jnp.float32),
    mesh=_mesh,
    scratch_types=[
        pltpu.VMEM((2, 4, C2), jnp.int32),
        pltpu.VMEM((2, C2), jnp.int32),
        pltpu.VMEM((2, 4, C2), jnp.int32),
        pltpu.VMEM((2, 4, C2), jnp.float32),
        pltpu.VMEM((2, 4 * C2, 64), jnp.float32),
        pltpu.VMEM((C2, 80), jnp.float32),
        pltpu.VMEM_SHARED((N2_PAD, 80), jnp.float32),
        pltpu.SemaphoreType.DMA,
        pltpu.SemaphoreType.DMA,
    ],
    compiler_params=_sc_params,
)


def _elu(v):
    return jnp.where(v > 0, v, jnp.exp(jnp.minimum(v, 0.0)) - 1.0)


def _tca1_body(acc_ref, x_ref, w1r_ref, root1_ref, bias1_ref, h_ref):
    a = acc_ref[0]
    deg = jnp.maximum(a[:, 25:26], 1.0)
    out = jnp.dot(a[:, :25], w1r_ref[...],
                  preferred_element_type=jnp.float32) / deg
    out = out + x_ref[...] * root1_ref[...] + bias1_ref[...][None, :]
    out = _elu(out)
    p = out.reshape(14, 2, 14, 2, 32).max(axis=(1, 3))
    h_ref[...] = p.reshape(1, 196, 32)


def _tca2_body(h_ref, w_ref, o_ref):
    o_ref[...] = jnp.dot(h_ref[...], w_ref[...],
                         preferred_element_type=jnp.float32)


def _tcb_body(acc_ref, h_ref, root2_ref, bias2_ref, o_ref):
    a = acc_ref[0, :N2] + acc_ref[1, :N2]
    deg = jnp.maximum(a[:, 64:65], 1.0)
    out = a[:, :64] / deg
    out = out + jnp.dot(h_ref[...], root2_ref[...],
                        preferred_element_type=jnp.float32)
    out = _elu(out + bias2_ref[...][None, :])
    p = out.reshape(64, 7, 2, 7, 2, 64).max(axis=(2, 4))
    o_ref[...] = p.reshape(3136, 64)


def _tcc_body(inp_ref, w1_ref, b1_ref, w2_ref, b2_ref, o_ref):
    z = jnp.dot(inp_ref[...], w1_ref[...],
                preferred_element_type=jnp.float32) + b1_ref[...][None, :]
    z = _elu(z)
    z = jnp.dot(z, w2_ref[...],
                preferred_element_type=jnp.float32) + b2_ref[...][None, :]
    z = _elu(z)
    m = jnp.max(z, axis=-1, keepdims=True)
    lse = m + jnp.log(jnp.sum(jnp.exp(z - m), axis=-1, keepdims=True))
    o_ref[...] = z - lse


def _pack_edges(ei, ps, cw):
    src = ei[0].astype(jnp.int32)
    dst = ei[1].astype(jnp.int32)
    p0 = lax.bitcast_convert_type(ps[:, 0], jnp.int32)
    p1 = lax.bitcast_convert_type(ps[:, 1], jnp.int32)
    n = src.shape[0] // cw
    return jnp.stack([src.reshape(n, cw), dst.reshape(n, cw),
                      p0.reshape(n, cw), p1.reshape(n, cw)], axis=1)


def kernel(x, edge_index1, pseudo1, edge_index2, pseudo2,
           W1, root1, bias1, W2, root2, bias2, fc1_w, fc1_b, fc2_w, fc2_b):
    f32 = jnp.float32
    xf = x[:, 0]

    acc1 = _conv1_edges(_pack_edges(edge_index1, pseudo1, C1), xf)

    h = pl.pallas_call(
        _tca1_body,
        grid=(64,),
        in_specs=[
            pl.BlockSpec((1, 784, 32), lambda i: (i // 32, i % 32, 0)),
            pl.BlockSpec((784, 1), lambda i: (i, 0)),
            pl.BlockSpec((25, 32), lambda i: (0, 0)),
            pl.BlockSpec((1, 32), lambda i: (0, 0)),
            pl.BlockSpec((32,), lambda i: (0,)),
        ],
        out_specs=pl.BlockSpec((1, 196, 32), lambda i: (i, 0, 0)),
        out_shape=jax.ShapeDtypeStruct((64, 196, 32), f32),
    )(acc1, x, W1[:, 0, :], root1, bias1)
    h = h.reshape(N2, 32)

    W2f = W2.transpose(1, 0, 2).reshape(32, 25 * 64)
    t2 = pl.pallas_call(
        _tca2_body,
        grid=(49,),
        in_specs=[
            pl.BlockSpec((256, 32), lambda i: (i, 0)),
            pl.BlockSpec((32, 1600), lambda i: (0, 0)),
        ],
        out_specs=pl.BlockSpec((256, 1600), lambda i: (i, 0)),
        out_shape=jax.ShapeDtypeStruct((N2, 1600), f32),
    )(h, W2f)
    t2 = t2.reshape(N2 * 25, 64)

    acc2 = _conv2_edges(_pack_edges(edge_index2, pseudo2, C2), t2)

    pooled = pl.pallas_call(
        _tcb_body,
        in_specs=[
            pl.BlockSpec((2, N2_PAD, 80), lambda: (0, 0, 0)),
            pl.BlockSpec((N2, 32), lambda: (0, 0)),
            pl.BlockSpec((32, 64), lambda: (0, 0)),
            pl.BlockSpec((64,), lambda: (0,)),
        ],
        out_specs=pl.BlockSpec((3136, 64), lambda: (0, 0)),
        out_shape=jax.ShapeDtypeStruct((3136, 64), f32),
    )(acc2, h, root2, bias2)

    out = pl.pallas_call(
        _tcc_body,
        in_specs=[
            pl.BlockSpec((64, 3136), lambda: (0, 0)),
            pl.BlockSpec((3136, 512), lambda: (0, 0)),
            pl.BlockSpec((512,), lambda: (0,)),
            pl.BlockSpec((512, 10), lambda: (0, 0)),
            pl.BlockSpec((10,), lambda: (0,)),
        ],
        out_specs=pl.BlockSpec((64, 10), lambda: (0, 0)),
        out_shape=jax.ShapeDtypeStruct((64, 10), f32),
    )(pooled.reshape(64, 3136), fc1_w, fc1_b, fc2_w, fc2_b)
    return out


# speedup vs baseline: 5.0297x; 1.0031x over previous
import jax
import jax.numpy as jnp
from jax import lax
from jax.experimental import pallas as pl
from jax.experimental.pallas import tpu as pltpu
from jax.experimental.pallas import tpu_sc as plsc

K = 5
NC, NS, LANES = 2, 16, 16
NTILES = NC * NS

N1 = 64 * 28 * 28
N2 = 64 * 14 * 14
E1 = N1 * 8
E2 = N2 * 8

C1 = 128
C2 = 64
N1H = N1 // 2
N1H_PAD = 25600
EPT1 = E1 // NS
NCH1 = EPT1 // C1
EPT2 = E2 // NTILES
NCH2 = EPT2 // C2
N2_PAD = 12800

_mesh = plsc.VectorSubcoreMesh(core_axis_name="c", subcore_axis_name="s")
_sc_params = pltpu.CompilerParams(needs_layout_passes=False,
                                  use_tc_tiling_on_sc=False)


def _conv1_edges_body(ed_hbm, x_hbm, out_hbm,
                      x_v, ebuf, dstb, rowbuf, agg,
                      esem0, esem1, ssem0, ssem1):
    esems = (esem0, esem1)
    ssems = (ssem0, ssem1)
    c = lax.axis_index("c")
    s = lax.axis_index("s")
    zeros16 = jnp.zeros((LANES,), jnp.float32)
    cbase = s * NCH1
    maxc = NS * NCH1 - 1
    dst_lo = c * N1H
    iota16 = lax.iota(jnp.int32, 16)
    col_deg = jnp.full((16,), 25, jnp.int32)
    ones16 = jnp.ones((16,), jnp.float32)

    def _ed_issue(cidx, b):
        pltpu.async_copy(ed_hbm.at[jnp.minimum(cbase + cidx, maxc)],
                         ebuf.at[b], esems[b])

    def _ed_wait(b):
        pltpu.make_async_copy(ed_hbm.at[0], ebuf.at[b], esems[b]).wait()

    def _s_issue(b):
        pltpu.async_copy(rowbuf.at[b], agg.at[dstb.at[b]], ssems[b], add=True)

    def _s_wait(b):
        pltpu.make_async_copy(rowbuf.at[b], agg.at[dstb.at[b]],
                              ssems[b]).wait()

    _ed_issue(0, 0)
    pltpu.sync_copy(x_hbm, x_v)

    def _zero_rowbuf(b):
        def _zr(i, carry):
            for r4 in range(4):
                rowbuf[b, i * 4 + r4, pl.ds(0, 16)] = zeros16
                rowbuf[b, i * 4 + r4, pl.ds(16, 16)] = zeros16
            return carry
        lax.fori_loop(0, C1 // 4, _zr, 0)

    _zero_rowbuf(0)
    _zero_rowbuf(1)

    def _zero_agg(i, carry):
        pltpu.sync_copy(rowbuf.at[0, pl.ds(0, 64)],
                        agg.at[pl.ds(s * 1600 + i * 64, 64)])
        return carry
    lax.fori_loop(0, 25, _zero_agg, 0)
    plsc.subcore_barrier()

    def _do_v(b):
        def _vec(j, carry):
            sl = pl.ds(j * LANES, LANES)
            sv = ebuf[b, 0, sl]
            dv = ebuf[b, 1, sl] - dst_lo
            dv = jnp.where((dv >= 0) & (dv < N1H), dv, N1H)
            dstb[b, sl] = dv
            a0 = plsc.bitcast(ebuf[b, 2, sl], jnp.float32) * (K - 1.0)
            a1 = plsc.bitcast(ebuf[b, 3, sl], jnp.float32) * (K - 1.0)
            i0 = a0.astype(jnp.int32)
            i1 = a1.astype(jnp.int32)
            fr0 = a0 - i0.astype(jnp.float32)
            fr1 = a1 - i1.astype(jnp.float32)
            xs = plsc.load_gather(x_v, [sv])
            rowv = j * LANES + iota16
            rb = rowbuf.at[b]
            for c0 in (1, 0):
                k0 = jnp.clip(i0 + c0, 0, K - 1)
                w0 = fr0 if c0 else 1.0 - fr0
                for c1 in (1, 0):
                    k1 = jnp.clip(i1 + c1, 0, K - 1)
                    w1 = fr1 if c1 else 1.0 - fr1
                    plsc.store_scatter(rb, [rowv, k0 * K + k1],
                                       (w0 * w1) * xs)
            plsc.store_scatter(rb, [rowv, col_deg], ones16)
            return carry
        lax.fori_loop(0, C1 // LANES, _vec, 0)

    _ed_wait(0)
    _ed_issue(1, 1)
    _do_v(0)
    _ed_issue(2, 0)
    _s_issue(0)
    _ed_wait(1)
    _do_v(1)
    _ed_issue(3, 1)
    _s_issue(1)

    def _main(k, carry):
        for half in (0, 1):
            ci = 2 + 2 * k + half
            b = half
            _ed_wait(b)
            _s_wait(b)
            _zero_rowbuf(b)
            _do_v(b)
            _ed_issue(ci + 2, b)
            _s_issue(b)
        return carry
    lax.fori_loop(0, (NCH1 - 2) // 2, _main, 0)

    _s_wait(0)
    _s_wait(1)
    _ed_wait(0)
    _ed_wait(1)

    plsc.subcore_barrier()
    pltpu.sync_copy(agg.at[pl.ds(s * 1600, 1600)],
                    out_hbm.at[c, pl.ds(s * 1600, 1600)])


_conv1_edges = pl.kernel(
    _conv1_edges_body,
    out_type=jax.ShapeDtypeStruct((NC, N1H_PAD, 32), jnp.float32),
    mesh=_mesh,
    scratch_types=[
        pltpu.VMEM((N1,), jnp.float32),
        pltpu.VMEM((2, 4, C1), jnp.int32),
        pltpu.VMEM((2, C1), jnp.int32),
        pltpu.VMEM((2, C1, 32), jnp.float32),
        pltpu.VMEM_SHARED((N1H_PAD, 32), jnp.float32),
        pltpu.SemaphoreType.DMA,
        pltpu.SemaphoreType.DMA,
        pltpu.SemaphoreType.DMA,
        pltpu.SemaphoreType.DMA,
    ],
    compiler_params=_sc_params,
)


def _conv2_edges_body(ed_hbm, t2_hbm, out_hbm,
                      ebuf, dstb, sbidx, gbuf, wbuf, rbuf, msgbuf, agg,
                      gsem0, gsem1, ssem0, ssem1, esem):
    gsems = (gsem0, gsem1)
    ssems = (ssem0, ssem1)
    c = lax.axis_index("c")
    s = lax.axis_index("s")
    tid = c * NS + s
    zeros16 = jnp.zeros((LANES,), jnp.float32)
    iota16 = lax.iota(jnp.int32, 16)
    cbase = tid * NCH2
    maxc = NTILES * NCH2 - 1

    def _zero_msgbuf(i, carry):
        for hh in range(5):
            msgbuf[0, i, pl.ds(hh * 16, 16)] = zeros16
            msgbuf[1, i, pl.ds(hh * 16, 16)] = zeros16
        return carry
    lax.fori_loop(0, C2, _zero_msgbuf, 0)

    def _zero_agg(i, carry):
        pltpu.sync_copy(msgbuf.at[0, pl.ds(0, 50)],
                        agg.at[pl.ds(s * 800 + i * 50, 50)])
        return carry
    lax.fori_loop(0, 16, _zero_agg, 0)
    plsc.subcore_barrier()

    deg16 = jnp.where(iota16 == 0, 1.0, 0.0).astype(jnp.float32)

    def _init_deg(i, carry):
        msgbuf[0, i, pl.ds(64, 16)] = deg16
        msgbuf[1, i, pl.ds(64, 16)] = deg16
        return carry
    lax.fori_loop(0, C2, _init_deg, 0)

    def _ed_issue(cidx, b):
        pltpu.async_copy(ed_hbm.at[jnp.minimum(cbase + cidx, maxc)],
                         ebuf.at[b], esem)

    def _ed_wait(b):
        pltpu.make_async_copy(ed_hbm.at[0], ebuf.at[b], esem).wait()

    def _do_v(b):
        def _vec(j, carry):
            sl = pl.ds(j * LANES, LANES)
            sv = ebuf[b, 0, sl]
            dstb[b, sl] = ebuf[b, 1, sl]
            a0 = plsc.bitcast(ebuf[b, 2, sl], jnp.float32) * (K - 1.0)
            a1 = plsc.bitcast(ebuf[b, 3, sl], jnp.float32) * (K - 1.0)
            i0 = a0.astype(jnp.int32)
            i1 = a1.astype(jnp.int32)
            fr0 = a0 - i0.astype(jnp.float32)
            fr1 = a1 - i1.astype(jnp.float32)
            sv25 = sv * 25
            ix = 0
            for c0 in (0, 1):
                k0 = jnp.clip(i0 + c0, 0, K - 1)
                w0 = fr0 if c0 else 1.0 - fr0
                for c1 in (0, 1):
                    k1 = jnp.clip(i1 + c1, 0, K - 1)
                    w1 = fr1 if c1 else 1.0 - fr1
                    gbuf[b, ix, sl] = sv25 + k0 * K + k1
                    wbuf[b, ix, sl] = w0 * w1
                    ix += 1
            return carry
        lax.fori_loop(0, C2 // LANES, _vec, 0)

    def _g_issue(b):
        for cc in range(4):
            pltpu.async_copy(t2_hbm.at[gbuf.at[b, cc]],
                             rbuf.at[b, pl.ds(cc * C2, C2)], gsems[b])

    def _g_wait(b):
        for cc in range(4):
            pltpu.make_async_copy(t2_hbm.at[gbuf.at[b, cc]],
                                  rbuf.at[b, pl.ds(cc * C2, C2)],
                                  gsems[b]).wait()

    def _s_issue(b):
        for q in range(C2 // LANES):
            qq = pl.ds(q * LANES, LANES)
            sbidx[b, qq] = dstb[b, qq]
        pltpu.async_copy(msgbuf.at[b], agg.at[sbidx.at[b]], ssems[b],
                         add=True)

    def _s_wait(b):
        pltpu.make_async_copy(msgbuf.at[b], agg.at[sbidx.at[b]],
                              ssems[b]).wait()

    def _do_m(b):
        rb = rbuf.at[b]
        mb = msgbuf.at[b]

        def _mgrp(j, carry):
            ev = j * LANES + iota16
            slw = pl.ds(j * LANES, LANES)
            wv0 = wbuf[b, 0, slw]
            wv1 = wbuf[b, 1, slw]
            wv2 = wbuf[b, 2, slw]
            wv3 = wbuf[b, 3, slw]
            for o in range(64):
                ov = jnp.full((16,), o, jnp.int32)
                m = (wv0 * plsc.load_gather(rb, [ev, ov])
                     + wv1 * plsc.load_gather(rb, [ev + C2, ov])
                     + wv2 * plsc.load_gather(rb, [ev + 2 * C2, ov])
                     + wv3 * plsc.load_gather(rb, [ev + 3 * C2, ov]))
                plsc.store_scatter(mb, [ev, ov], m)
            return carry
        lax.fori_loop(0, C2 // LANES, _mgrp, 0)

    _ed_issue(0, 0)
    _ed_wait(0)
    _do_v(0)
    _g_issue(0)
    _ed_issue(1, 1)

    def _half(ci, b, swait):
        _ed_wait(b)
        _do_v(b)
        _g_issue(b)
        _ed_issue(ci + 1, 1 - b)
        _g_wait(1 - b)
        if swait:
            _s_wait(1 - b)
        _do_m(1 - b)
        _s_issue(1 - b)

    _half(1, 1, False)
    _half(2, 0, False)

    def _main(k, carry):
        for half in (0, 1):
            ci = 3 + 2 * k + half
            b = 1 - half
            _half(ci, b, True)
        return carry
    lax.fori_loop(0, (NCH2 - 3) // 2, _main, 0)

    _g_wait(0)
    _s_wait(0)
    _do_m(0)
    _s_issue(0)
    _s_wait(1)
    _s_wait(0)
    _ed_wait(1)

    plsc.subcore_barrier()
    pltpu.sync_copy(agg.at[pl.ds(s * 800, 800)],
                    out_hbm.at[c, pl.ds(s * 800, 800)])


_conv2_edges = pl.kernel(
    _conv2_edges_body,
    out_type=jax.ShapeDtypeStruct((NC, N2_PAD, 80), jnp.float32),
    mesh=_mesh,
    scratch_types=[
        pltpu.VMEM((2, 4, C2), jnp.int32),
        pltpu.VMEM((2, C2), jnp.int32),
        pltpu.VMEM((2, C2), jnp.int32),
        pltpu.VMEM((2, 4, C2), jnp.int32),
        pltpu.VMEM((2, 4, C2), jnp.float32),
        pltpu.VMEM((2, 4 * C2, 64), jnp.float32),
        pltpu.VMEM((2, C2, 80), jnp.float32),
        pltpu.VMEM_SHARED((N2_PAD, 80), jnp.float32),
        pltpu.SemaphoreType.DMA,
        pltpu.SemaphoreType.DMA,
        pltpu.SemaphoreType.DMA,
        pltpu.SemaphoreType.DMA,
        pltpu.SemaphoreType.DMA,
    ],
    compiler_params=_sc_params,
)


def _elu(v):
    return jnp.where(v > 0, v, jnp.exp(jnp.minimum(v, 0.0)) - 1.0)


def _tca1_body(acc_ref, x_ref, w1r_ref, root1_ref, bias1_ref, h_ref):
    a = acc_ref[0]
    deg = jnp.maximum(a[:, 25:26], 1.0)
    out = jnp.dot(a[:, :25], w1r_ref[...],
                  preferred_element_type=jnp.float32) / deg
    out = out + x_ref[...] * root1_ref[...] + bias1_ref[...][None, :]
    out = _elu(out)
    p = out.reshape(14, 2, 14, 2, 32).max(axis=(1, 3))
    h_ref[...] = p.reshape(1, 196, 32)


def _tca2_body(h_ref, w_ref, o_ref):
    o_ref[...] = jnp.dot(h_ref[...], w_ref[...],
                         preferred_element_type=jnp.float32)


def _tcb_body(acc_ref, h_ref, root2_ref, bias2_ref, o_ref):
    a = acc_ref[0, :N2] + acc_ref[1, :N2]
    deg = jnp.maximum(a[:, 64:65], 1.0)
    out = a[:, :64] / deg
    out = out + jnp.dot(h_ref[...], root2_ref[...],
                        preferred_element_type=jnp.float32)
    out = _elu(out + bias2_ref[...][None, :])
    p = out.reshape(64, 7, 2, 7, 2, 64).max(axis=(2, 4))
    o_ref[...] = p.reshape(3136, 64)


def _tcc_body(inp_ref, w1_ref, b1_ref, w2_ref, b2_ref, o_ref):
    z = jnp.dot(inp_ref[...], w1_ref[...],
                preferred_element_type=jnp.float32) + b1_ref[...][None, :]
    z = _elu(z)
    z = jnp.dot(z, w2_ref[...],
                preferred_element_type=jnp.float32) + b2_ref[...][None, :]
    z = _elu(z)
    m = jnp.max(z, axis=-1, keepdims=True)
    lse = m + jnp.log(jnp.sum(jnp.exp(z - m), axis=-1, keepdims=True))
    o_ref[...] = z - lse


def _pack_edges(ei, ps, cw):
    src = ei[0].astype(jnp.int32)
    dst = ei[1].astype(jnp.int32)
    p0 = lax.bitcast_convert_type(ps[:, 0], jnp.int32)
    p1 = lax.bitcast_convert_type(ps[:, 1], jnp.int32)
    n = src.shape[0] // cw
    return jnp.stack([src.reshape(n, cw), dst.reshape(n, cw),
                      p0.reshape(n, cw), p1.reshape(n, cw)], axis=1)


def kernel(x, edge_index1, pseudo1, edge_index2, pseudo2,
           W1, root1, bias1, W2, root2, bias2, fc1_w, fc1_b, fc2_w, fc2_b):
    f32 = jnp.float32
    xf = x[:, 0]

    acc1 = _conv1_edges(_pack_edges(edge_index1, pseudo1, C1), xf)

    h = pl.pallas_call(
        _tca1_body,
        grid=(64,),
        in_specs=[
            pl.BlockSpec((1, 784, 32), lambda i: (i // 32, i % 32, 0)),
            pl.BlockSpec((784, 1), lambda i: (i, 0)),
            pl.BlockSpec((25, 32), lambda i: (0, 0)),
            pl.BlockSpec((1, 32), lambda i: (0, 0)),
            pl.BlockSpec((32,), lambda i: (0,)),
        ],
        out_specs=pl.BlockSpec((1, 196, 32), lambda i: (i, 0, 0)),
        out_shape=jax.ShapeDtypeStruct((64, 196, 32), f32),
    )(acc1, x, W1[:, 0, :], root1, bias1)
    h = h.reshape(N2, 32)

    W2f = W2.transpose(1, 0, 2).reshape(32, 25 * 64)
    t2 = pl.pallas_call(
        _tca2_body,
        grid=(49,),
        in_specs=[
            pl.BlockSpec((256, 32), lambda i: (i, 0)),
            pl.BlockSpec((32, 1600), lambda i: (0, 0)),
        ],
        out_specs=pl.BlockSpec((256, 1600), lambda i: (i, 0)),
        out_shape=jax.ShapeDtypeStruct((N2, 1600), f32),
    )(h, W2f)
    t2 = t2.reshape(N2 * 25, 64)

    acc2 = _conv2_edges(_pack_edges(edge_index2, pseudo2, C2), t2)

    pooled = pl.pallas_call(
        _tcb_body,
        in_specs=[
            pl.BlockSpec((2, N2_PAD, 80), lambda: (0, 0, 0)),
            pl.BlockSpec((N2, 32), lambda: (0, 0)),
            pl.BlockSpec((32, 64), lambda: (0, 0)),
            pl.BlockSpec((64,), lambda: (0,)),
        ],
        out_specs=pl.BlockSpec((3136, 64), lambda: (0, 0)),
        out_shape=jax.ShapeDtypeStruct((3136, 64), f32),
    )(acc2, h, root2, bias2)

    out = pl.pallas_call(
        _tcc_body,
        in_specs=[
            pl.BlockSpec((64, 3136), lambda: (0, 0)),
            pl.BlockSpec((3136, 512), lambda: (0, 0)),
            pl.BlockSpec((512,), lambda: (0,)),
            pl.BlockSpec((512, 10), lambda: (0, 0)),
            pl.BlockSpec((10,), lambda: (0,)),
        ],
        out_specs=pl.BlockSpec((64, 10), lambda: (0, 0)),
        out_shape=jax.ShapeDtypeStruct((64, 10), f32),
    )(pooled.reshape(64, 3136), fc1_w, fc1_b, fc2_w, fc2_b)
    return out


# speedup vs baseline: 5.4145x; 1.0765x over previous
import jax
import jax.numpy as jnp
from jax import lax
from jax.experimental import pallas as pl
from jax.experimental.pallas import tpu as pltpu
from jax.experimental.pallas import tpu_sc as plsc

K = 5
NC, NS, LANES = 2, 16, 16
NTILES = NC * NS

N1 = 64 * 28 * 28
N2 = 64 * 14 * 14
E1 = N1 * 8
E2 = N2 * 8

C1 = 128
C2 = 64
N1H = N1 // 2
N1H_PAD = 25600
EPT1 = E1 // NS
NCH1 = EPT1 // C1
EPT2 = E2 // NTILES
NCH2 = EPT2 // C2
N2_PAD = 12800

_mesh = plsc.VectorSubcoreMesh(core_axis_name="c", subcore_axis_name="s")
_sc_params = pltpu.CompilerParams(needs_layout_passes=False,
                                  use_tc_tiling_on_sc=False)


def _conv1_edges_body(ed_hbm, x_hbm, out_hbm,
                      x_v, ebuf, dstb, rowbuf, agg,
                      esem0, esem1, ssem0, ssem1):
    esems = (esem0, esem1)
    ssems = (ssem0, ssem1)
    c = lax.axis_index("c")
    s = lax.axis_index("s")
    zeros16 = jnp.zeros((LANES,), jnp.float32)
    cbase = s * NCH1
    maxc = NS * NCH1 - 1
    dst_lo = c * N1H
    iota16 = lax.iota(jnp.int32, 16)
    col_deg = jnp.full((16,), 25, jnp.int32)
    ones16 = jnp.ones((16,), jnp.float32)

    def _ed_issue(cidx, b):
        pltpu.async_copy(ed_hbm.at[jnp.minimum(cbase + cidx, maxc)],
                         ebuf.at[b], esems[b])

    def _ed_wait(b):
        pltpu.make_async_copy(ed_hbm.at[0], ebuf.at[b], esems[b]).wait()

    def _s_issue(b):
        pltpu.async_copy(rowbuf.at[b], agg.at[dstb.at[b]], ssems[b], add=True)

    def _s_wait(b):
        pltpu.make_async_copy(rowbuf.at[b], agg.at[dstb.at[b]],
                              ssems[b]).wait()

    _ed_issue(0, 0)
    pltpu.sync_copy(x_hbm, x_v)

    def _zero_rowbuf(b):
        def _zr(i, carry):
            for r4 in range(4):
                rowbuf[b, i * 4 + r4, pl.ds(0, 16)] = zeros16
                rowbuf[b, i * 4 + r4, pl.ds(16, 16)] = zeros16
            return carry
        lax.fori_loop(0, C1 // 4, _zr, 0)

    _zero_rowbuf(0)
    _zero_rowbuf(1)

    def _zero_agg(i, carry):
        pltpu.sync_copy(rowbuf.at[0, pl.ds(0, 64)],
                        agg.at[pl.ds(s * 1600 + i * 64, 64)])
        return carry
    lax.fori_loop(0, 25, _zero_agg, 0)
    plsc.subcore_barrier()

    def _do_v(b):
        def _vec(j, carry):
            sl = pl.ds(j * LANES, LANES)
            sv = ebuf[b, 0, sl]
            dv = ebuf[b, 1, sl] - dst_lo
            dv = jnp.where((dv >= 0) & (dv < N1H), dv, N1H)
            dstb[b, sl] = dv
            a0 = plsc.bitcast(ebuf[b, 2, sl], jnp.float32) * (K - 1.0)
            a1 = plsc.bitcast(ebuf[b, 3, sl], jnp.float32) * (K - 1.0)
            i0 = a0.astype(jnp.int32)
            i1 = a1.astype(jnp.int32)
            fr0 = a0 - i0.astype(jnp.float32)
            fr1 = a1 - i1.astype(jnp.float32)
            xs = plsc.load_gather(x_v, [sv])
            rowv = j * LANES + iota16
            rb = rowbuf.at[b]
            for c0 in (1, 0):
                k0 = jnp.clip(i0 + c0, 0, K - 1)
                w0 = fr0 if c0 else 1.0 - fr0
                for c1 in (1, 0):
                    k1 = jnp.clip(i1 + c1, 0, K - 1)
                    w1 = fr1 if c1 else 1.0 - fr1
                    plsc.store_scatter(rb, [rowv, k0 * K + k1],
                                       (w0 * w1) * xs)
            plsc.store_scatter(rb, [rowv, col_deg], ones16)
            return carry
        lax.fori_loop(0, C1 // LANES, _vec, 0)

    _ed_wait(0)
    _ed_issue(1, 1)
    _do_v(0)
    _ed_issue(2, 0)
    _s_issue(0)
    _ed_wait(1)
    _do_v(1)
    _ed_issue(3, 1)
    _s_issue(1)

    def _main(k, carry):
        for half in (0, 1):
            ci = 2 + 2 * k + half
            b = half
            _ed_wait(b)
            _s_wait(b)
            _zero_rowbuf(b)
            _do_v(b)
            _ed_issue(ci + 2, b)
            _s_issue(b)
        return carry
    lax.fori_loop(0, (NCH1 - 2) // 2, _main, 0)

    _s_wait(0)
    _s_wait(1)
    _ed_wait(0)
    _ed_wait(1)

    plsc.subcore_barrier()
    pltpu.sync_copy(agg.at[pl.ds(s * 1600, 1600)],
                    out_hbm.at[c, pl.ds(s * 1600, 1600)])


_conv1_edges = pl.kernel(
    _conv1_edges_body,
    out_type=jax.ShapeDtypeStruct((NC, N1H_PAD, 32), jnp.float32),
    mesh=_mesh,
    scratch_types=[
        pltpu.VMEM((N1,), jnp.float32),
        pltpu.VMEM((2, 4, C1), jnp.int32),
        pltpu.VMEM((2, C1), jnp.int32),
        pltpu.VMEM((2, C1, 32), jnp.float32),
        pltpu.VMEM_SHARED((N1H_PAD, 32), jnp.float32),
        pltpu.SemaphoreType.DMA,
        pltpu.SemaphoreType.DMA,
        pltpu.SemaphoreType.DMA,
        pltpu.SemaphoreType.DMA,
    ],
    compiler_params=_sc_params,
)


def _conv2_edges_body(ed_hbm, t2_hbm, out_hbm,
                      ebuf, dstb, sbidx, gbuf, wbuf, rbuf, msgbuf, agg,
                      gsem0, gsem1, ssem0, ssem1, esem):
    gsems = (gsem0, gsem1)
    ssems = (ssem0, ssem1)
    c = lax.axis_index("c")
    s = lax.axis_index("s")
    tid = c * NS + s
    zeros16 = jnp.zeros((LANES,), jnp.float32)
    iota16 = lax.iota(jnp.int32, 16)
    cbase = tid * NCH2
    maxc = NTILES * NCH2 - 1

    def _zero_msgbuf(i, carry):
        for hh in range(5):
            msgbuf[0, i, pl.ds(hh * 16, 16)] = zeros16
            msgbuf[1, i, pl.ds(hh * 16, 16)] = zeros16
        return carry
    lax.fori_loop(0, C2, _zero_msgbuf, 0)

    def _zero_agg(i, carry):
        pltpu.sync_copy(msgbuf.at[0, pl.ds(0, 50)],
                        agg.at[pl.ds(s * 800 + i * 50, 50)])
        return carry
    lax.fori_loop(0, 16, _zero_agg, 0)
    plsc.subcore_barrier()

    deg16 = jnp.where(iota16 == 0, 1.0, 0.0).astype(jnp.float32)

    def _init_deg(i, carry):
        msgbuf[0, i, pl.ds(64, 16)] = deg16
        msgbuf[1, i, pl.ds(64, 16)] = deg16
        return carry
    lax.fori_loop(0, C2, _init_deg, 0)

    def _ed_issue(cidx, b):
        pltpu.async_copy(ed_hbm.at[jnp.minimum(cbase + cidx, maxc)],
                         ebuf.at[b], esem)

    def _ed_wait(b):
        pltpu.make_async_copy(ed_hbm.at[0], ebuf.at[b], esem).wait()

    def _do_v(b):
        def _vec(j, carry):
            sl = pl.ds(j * LANES, LANES)
            sv = ebuf[b, 0, sl]
            dstb[b, sl] = ebuf[b, 1, sl]
            a0 = plsc.bitcast(ebuf[b, 2, sl], jnp.float32) * (K - 1.0)
            a1 = plsc.bitcast(ebuf[b, 3, sl], jnp.float32) * (K - 1.0)
            i0 = a0.astype(jnp.int32)
            i1 = a1.astype(jnp.int32)
            fr0 = a0 - i0.astype(jnp.float32)
            fr1 = a1 - i1.astype(jnp.float32)
            k1lo = jnp.clip(i1, 0, K - 1)
            for c0 in (0, 1):
                k0 = jnp.clip(i0 + c0, 0, K - 1)
                w0 = fr0 if c0 else 1.0 - fr0
                gbuf[b, c0, sl] = (k0 * K + k1lo) * N2 + sv
                wbuf[b, 2 * c0, sl] = w0 * (1.0 - fr1)
                wbuf[b, 2 * c0 + 1, sl] = w0 * fr1
            return carry
        lax.fori_loop(0, C2 // LANES, _vec, 0)

    def _g_issue(b):
        for cc in range(2):
            pltpu.async_copy(t2_hbm.at[gbuf.at[b, cc]],
                             rbuf.at[b, pl.ds(cc * C2, C2)], gsems[b])

    def _g_wait(b):
        for cc in range(2):
            pltpu.make_async_copy(t2_hbm.at[gbuf.at[b, cc]],
                                  rbuf.at[b, pl.ds(cc * C2, C2)],
                                  gsems[b]).wait()

    def _s_issue(b):
        for q in range(C2 // LANES):
            qq = pl.ds(q * LANES, LANES)
            sbidx[b, qq] = dstb[b, qq]
        pltpu.async_copy(msgbuf.at[b], agg.at[sbidx.at[b]], ssems[b],
                         add=True)

    def _s_wait(b):
        pltpu.make_async_copy(msgbuf.at[b], agg.at[sbidx.at[b]],
                              ssems[b]).wait()

    def _do_m(b):
        rb = rbuf.at[b]
        mb = msgbuf.at[b]

        def _mgrp(j, carry):
            ev = j * LANES + iota16
            slw = pl.ds(j * LANES, LANES)
            wv0 = wbuf[b, 0, slw]
            wv1 = wbuf[b, 1, slw]
            wv2 = wbuf[b, 2, slw]
            wv3 = wbuf[b, 3, slw]
            for o in range(64):
                ov = jnp.full((16,), o, jnp.int32)
                ov2 = jnp.full((16,), o + 64, jnp.int32)
                m = (wv0 * plsc.load_gather(rb, [ev, ov])
                     + wv1 * plsc.load_gather(rb, [ev, ov2])
                     + wv2 * plsc.load_gather(rb, [ev + C2, ov])
                     + wv3 * plsc.load_gather(rb, [ev + C2, ov2]))
                plsc.store_scatter(mb, [ev, ov], m)
            return carry
        lax.fori_loop(0, C2 // LANES, _mgrp, 0)

    _ed_issue(0, 0)
    _ed_wait(0)
    _do_v(0)
    _g_issue(0)
    _ed_issue(1, 1)

    def _half(ci, b, swait):
        _ed_wait(b)
        _do_v(b)
        _g_issue(b)
        _ed_issue(ci + 1, 1 - b)
        _g_wait(1 - b)
        if swait:
            _s_wait(1 - b)
        _do_m(1 - b)
        _s_issue(1 - b)

    _half(1, 1, False)
    _half(2, 0, False)

    def _main(k, carry):
        for half in (0, 1):
            ci = 3 + 2 * k + half
            b = 1 - half
            _half(ci, b, True)
        return carry
    lax.fori_loop(0, (NCH2 - 3) // 2, _main, 0)

    _g_wait(0)
    _s_wait(0)
    _do_m(0)
    _s_issue(0)
    _s_wait(1)
    _s_wait(0)
    _ed_wait(1)

    plsc.subcore_barrier()
    pltpu.sync_copy(agg.at[pl.ds(s * 800, 800)],
                    out_hbm.at[c, pl.ds(s * 800, 800)])


_conv2_edges = pl.kernel(
    _conv2_edges_body,
    out_type=jax.ShapeDtypeStruct((NC, N2_PAD, 80), jnp.float32),
    mesh=_mesh,
    scratch_types=[
        pltpu.VMEM((2, 4, C2), jnp.int32),
        pltpu.VMEM((2, C2), jnp.int32),
        pltpu.VMEM((2, C2), jnp.int32),
        pltpu.VMEM((2, 2, C2), jnp.int32),
        pltpu.VMEM((2, 4, C2), jnp.float32),
        pltpu.VMEM((2, 2 * C2, 128), jnp.float32),
        pltpu.VMEM((2, C2, 80), jnp.float32),
        pltpu.VMEM_SHARED((N2_PAD, 80), jnp.float32),
        pltpu.SemaphoreType.DMA,
        pltpu.SemaphoreType.DMA,
        pltpu.SemaphoreType.DMA,
        pltpu.SemaphoreType.DMA,
        pltpu.SemaphoreType.DMA,
    ],
    compiler_params=_sc_params,
)


def _elu(v):
    return jnp.where(v > 0, v, jnp.exp(jnp.minimum(v, 0.0)) - 1.0)


def _tca1_body(acc_ref, x_ref, w1r_ref, root1_ref, bias1_ref, h_ref):
    a = acc_ref[0]
    deg = jnp.maximum(a[:, 25:26], 1.0)
    out = jnp.dot(a[:, :25], w1r_ref[...],
                  preferred_element_type=jnp.float32) / deg
    out = out + x_ref[...] * root1_ref[...] + bias1_ref[...][None, :]
    out = _elu(out)
    p = out.reshape(14, 2, 14, 2, 32).max(axis=(1, 3))
    h_ref[...] = p.reshape(1, 196, 32)


def _tca2_body(h_ref, wa_ref, wb_ref, o_ref):
    t0 = jnp.dot(h_ref[...], wa_ref[0], preferred_element_type=jnp.float32)
    t1 = jnp.dot(h_ref[...], wb_ref[0], preferred_element_type=jnp.float32)
    o_ref[...] = jnp.concatenate([t0, t1], axis=-1)[None]


def _tcb_body(acc_ref, h_ref, root2_ref, bias2_ref, o_ref):
    a = acc_ref[0, :N2] + acc_ref[1, :N2]
    deg = jnp.maximum(a[:, 64:65], 1.0)
    out = a[:, :64] / deg
    out = out + jnp.dot(h_ref[...], root2_ref[...],
                        preferred_element_type=jnp.float32)
    out = _elu(out + bias2_ref[...][None, :])
    p = out.reshape(64, 7, 2, 7, 2, 64).max(axis=(2, 4))
    o_ref[...] = p.reshape(3136, 64)


def _tcc_body(inp_ref, w1_ref, b1_ref, w2_ref, b2_ref, o_ref):
    z = jnp.dot(inp_ref[...], w1_ref[...],
                preferred_element_type=jnp.float32) + b1_ref[...][None, :]
    z = _elu(z)
    z = jnp.dot(z, w2_ref[...],
                preferred_element_type=jnp.float32) + b2_ref[...][None, :]
    z = _elu(z)
    m = jnp.max(z, axis=-1, keepdims=True)
    lse = m + jnp.log(jnp.sum(jnp.exp(z - m), axis=-1, keepdims=True))
    o_ref[...] = z - lse


def _pack_edges(ei, ps, cw):
    src = ei[0].astype(jnp.int32)
    dst = ei[1].astype(jnp.int32)
    p0 = lax.bitcast_convert_type(ps[:, 0], jnp.int32)
    p1 = lax.bitcast_convert_type(ps[:, 1], jnp.int32)
    n = src.shape[0] // cw
    return jnp.stack([src.reshape(n, cw), dst.reshape(n, cw),
                      p0.reshape(n, cw), p1.reshape(n, cw)], axis=1)


def kernel(x, edge_index1, pseudo1, edge_index2, pseudo2,
           W1, root1, bias1, W2, root2, bias2, fc1_w, fc1_b, fc2_w, fc2_b):
    f32 = jnp.float32
    xf = x[:, 0]

    acc1 = _conv1_edges(_pack_edges(edge_index1, pseudo1, C1), xf)

    h = pl.pallas_call(
        _tca1_body,
        grid=(64,),
        in_specs=[
            pl.BlockSpec((1, 784, 32), lambda i: (i // 32, i % 32, 0)),
            pl.BlockSpec((784, 1), lambda i: (i, 0)),
            pl.BlockSpec((25, 32), lambda i: (0, 0)),
            pl.BlockSpec((1, 32), lambda i: (0, 0)),
            pl.BlockSpec((32,), lambda i: (0,)),
        ],
        out_specs=pl.BlockSpec((1, 196, 32), lambda i: (i, 0, 0)),
        out_shape=jax.ShapeDtypeStruct((64, 196, 32), f32),
    )(acc1, x, W1[:, 0, :], root1, bias1)
    h = h.reshape(N2, 32)

    t2 = pl.pallas_call(
        _tca2_body,
        grid=(25,),
        in_specs=[
            pl.BlockSpec((N2, 32), lambda k: (0, 0)),
            pl.BlockSpec((1, 32, 64), lambda k: (k, 0, 0)),
            pl.BlockSpec((1, 32, 64), lambda k: (jnp.minimum(k + 1, 24), 0, 0)),
        ],
        out_specs=pl.BlockSpec((1, N2, 128), lambda k: (k, 0, 0)),
        out_shape=jax.ShapeDtypeStruct((25, N2, 128), f32),
    )(h, W2, W2)
    t2 = t2.reshape(25 * N2, 128)

    acc2 = _conv2_edges(_pack_edges(edge_index2, pseudo2, C2), t2)

    pooled = pl.pallas_call(
        _tcb_body,
        in_specs=[
            pl.BlockSpec((2, N2_PAD, 80), lambda: (0, 0, 0)),
            pl.BlockSpec((N2, 32), lambda: (0, 0)),
            pl.BlockSpec((32, 64), lambda: (0, 0)),
            pl.BlockSpec((64,), lambda: (0,)),
        ],
        out_specs=pl.BlockSpec((3136, 64), lambda: (0, 0)),
        out_shape=jax.ShapeDtypeStruct((3136, 64), f32),
    )(acc2, h, root2, bias2)

    out = pl.pallas_call(
        _tcc_body,
        in_specs=[
            pl.BlockSpec((64, 3136), lambda: (0, 0)),
            pl.BlockSpec((3136, 512), lambda: (0, 0)),
            pl.BlockSpec((512,), lambda: (0,)),
            pl.BlockSpec((512, 10), lambda: (0, 0)),
            pl.BlockSpec((10,), lambda: (0,)),
        ],
        out_specs=pl.BlockSpec((64, 10), lambda: (0, 0)),
        out_shape=jax.ShapeDtypeStruct((64, 10), f32),
    )(pooled.reshape(64, 3136), fc1_w, fc1_b, fc2_w, fc2_b)
    return out
